# Initial kernel scaffold; baseline (speedup 1.0000x reference)
import jax
import jax.numpy as jnp
from jax.experimental import pallas as pl


def kernel(zi, y, latent, W_v, b_v, W_p, b_p, E_m):
    raise NotImplementedError("write your pallas kernel here")


# speedup vs baseline: 2.5336x; 2.5336x over previous
import functools

import jax
import jax.numpy as jnp
from jax.experimental import pallas as pl

_B = 64
_T = 16
_S = 10
_D = 128
_G = 1000
_GP = 1024
_ST = 64
_TOT = 4096
_R = _B * _S

_HI = jax.lax.Precision.HIGHEST


def _dot(a, b):
    return jax.lax.dot_general(a, b, (((a.ndim - 1,), (0,)), ((), ())),
                               precision=_HI, preferred_element_type=jnp.float32)


def _cumsum_lanes(x):
    r, n = x.shape
    d = 1
    while d < n:
        x = x + jnp.concatenate(
            [jnp.zeros((r, d), x.dtype), x[:, :-d]], axis=1)
        d *= 2
    return x


def _sample_body(zi_ref, latent_ref, rep_ref, wp_ref, bp_ref, em_ref, rd_ref,
                 zs_ref):
    iota_tot = jax.lax.broadcasted_iota(jnp.int32, (_B, _TOT), 1)
    onehot_zi = (zi_ref[...] == iota_tot).astype(jnp.float32)
    z0 = _dot(onehot_zi, latent_ref[...])
    z = _dot(rep_ref[...], z0)
    zold = jnp.zeros_like(z)
    iota_st = jax.lax.broadcasted_iota(jnp.int32, (_R, _ST), 1)
    for i in range(_T):
        mean = jnp.mean(z, axis=-1, keepdims=True)
        c = z - mean
        var = jnp.sum(c * c, axis=-1, keepdims=True) * (1.0 / (_D - 1))
        z = z / (1e-05 + jnp.sqrt(var)) * 0.113
        pref = _dot(jnp.concatenate([zold, z], axis=-1), wp_ref[...]) + bp_ref[...]
        m = jnp.max(pref, axis=-1, keepdims=True)
        e = jnp.exp(pref - m)
        xp = e / jnp.sum(e, axis=-1, keepdims=True)
        xpc = _cumsum_lanes(xp)
        rd = rd_ref[i]
        cnt = jnp.sum((xpc <= rd).astype(jnp.int32), axis=-1, keepdims=True)
        which = jnp.where(cnt >= _ST, 0, cnt)
        onehot = (which == iota_st).astype(jnp.float32)
        mvs = _dot(onehot, em_ref[...])
        zs_ref[i] = mvs
        zold = z
        z = z + mvs


def _proj_body(zs_ref, wv_ref, bv_ref, y_ref, pool_ref, out_ref):
    zrow = zs_ref[0]
    logits = _dot(zrow, wv_ref[...]) + bv_ref[...]
    m = jnp.max(logits, axis=-1, keepdims=True)
    lse = m + jnp.log(jnp.sum(jnp.exp(logits - m), axis=-1, keepdims=True))
    iota_g = jax.lax.broadcasted_iota(jnp.int32, (_R, _GP), 1)
    picked = jnp.sum(jnp.where(y_ref[0] == iota_g, logits, 0.0),
                     axis=-1, keepdims=True)
    yp = picked - lse
    grp = jax.lax.dot_general(yp, pool_ref[...], (((0,), (0,)), ((), ())),
                              precision=_HI,
                              preferred_element_type=jnp.float32)
    out_ref[0] = grp * (1.0 / _S)


def kernel(zi, y, latent, W_v, b_v, W_p, b_p, E_m):
    zi2 = zi.astype(jnp.int32).reshape(_B, 1)
    rep = (jnp.arange(_R)[:, None] // _S == jnp.arange(_B)[None, :]
           ).astype(jnp.float32)
    wp = W_p.T
    bp = b_p.reshape(1, _ST)
    rkey = jax.random.key(42)
    rd = jnp.stack([
        jax.random.uniform(jax.random.fold_in(rkey, i), (_B, _S),
                           dtype=jnp.float32).reshape(_R)
        for i in range(_T)
    ]).reshape(_T, _R, 1)

    zs = pl.pallas_call(
        _sample_body,
        out_shape=jax.ShapeDtypeStruct((_T, _R, _D), jnp.float32),
    )(zi2, latent, rep, wp, bp, E_m, rd)

    wv = jnp.zeros((_D, _GP), jnp.float32).at[:, :_G].set(W_v.T)
    bv = jnp.full((1, _GP), -1e30, jnp.float32).at[0, :_G].set(b_v)
    y_rep = jnp.repeat(y.astype(jnp.int32).T, _S, axis=1).reshape(_T, _R, 1)
    pool = rep

    out_t = pl.pallas_call(
        _proj_body,
        grid=(_T,),
        in_specs=[
            pl.BlockSpec((1, _R, _D), lambda t: (t, 0, 0)),
            pl.BlockSpec((_D, _GP), lambda t: (0, 0)),
            pl.BlockSpec((1, _GP), lambda t: (0, 0)),
            pl.BlockSpec((1, _R, 1), lambda t: (t, 0, 0)),
            pl.BlockSpec((_R, _B), lambda t: (0, 0)),
        ],
        out_specs=pl.BlockSpec((1, 1, _B), lambda t: (t, 0, 0)),
        out_shape=jax.ShapeDtypeStruct((_T, 1, _B), jnp.float32),
    )(zs, wv, bv, y_rep, pool)

    return out_t.reshape(_T, _B).T


# speedup vs baseline: 2.5478x; 1.0056x over previous
import jax
import jax.numpy as jnp
from jax.experimental import pallas as pl

_B = 64
_T = 16
_S = 10
_D = 128
_G = 1000
_GP = 1024
_ST = 64
_TOT = 4096
_R = _B * _S

_HI = jax.lax.Precision.HIGHEST


def _dot(a, b):
    return jax.lax.dot_general(a, b, (((a.ndim - 1,), (0,)), ((), ())),
                               precision=_HI, preferred_element_type=jnp.float32)


def _cumsum_lanes(x):
    r, n = x.shape
    d = 1
    while d < n:
        x = x + jnp.concatenate(
            [jnp.zeros((r, d), x.dtype), x[:, :-d]], axis=1)
        d *= 2
    return x


def _body(zi_ref, latent_ref, rep_ref, wp_ref, bp_ref, em_ref, rd_ref,
          wv_ref, bv_ref, y_ref, pool_ref, out_ref):
    iota_tot = jax.lax.broadcasted_iota(jnp.int32, (_B, _TOT), 1)
    onehot_zi = (zi_ref[...] == iota_tot).astype(jnp.float32)
    z0 = _dot(onehot_zi, latent_ref[...])
    z = _dot(rep_ref[...], z0)
    zold = jnp.zeros_like(z)
    iota_st = jax.lax.broadcasted_iota(jnp.int32, (_R, _ST), 1)
    iota_g = jax.lax.broadcasted_iota(jnp.int32, (_R, _GP), 1)
    for i in range(_T):
        mean = jnp.mean(z, axis=-1, keepdims=True)
        c = z - mean
        var = jnp.sum(c * c, axis=-1, keepdims=True) * (1.0 / (_D - 1))
        z = z / (1e-05 + jnp.sqrt(var)) * 0.113
        pref = _dot(jnp.concatenate([zold, z], axis=-1), wp_ref[...]) + bp_ref[...]
        m = jnp.max(pref, axis=-1, keepdims=True)
        e = jnp.exp(pref - m)
        xp = e / jnp.sum(e, axis=-1, keepdims=True)
        xpc = _cumsum_lanes(xp)
        rd = rd_ref[i]
        cnt = jnp.sum((xpc <= rd).astype(jnp.int32), axis=-1, keepdims=True)
        which = jnp.where(cnt >= _ST, 0, cnt)
        onehot = (which == iota_st).astype(jnp.float32)
        mvs = _dot(onehot, em_ref[...])

        logits = _dot(mvs, wv_ref[...]) + bv_ref[...]
        lm = jnp.max(logits, axis=-1, keepdims=True)
        lse = lm + jnp.log(jnp.sum(jnp.exp(logits - lm), axis=-1, keepdims=True))
        picked = jnp.sum(jnp.where(y_ref[i] == iota_g, logits, 0.0),
                         axis=-1, keepdims=True)
        yp = picked - lse
        grp = jax.lax.dot_general(yp, pool_ref[...], (((0,), (0,)), ((), ())),
                                  precision=_HI,
                                  preferred_element_type=jnp.float32)
        out_ref[i] = grp * (1.0 / _S)

        zold = z
        z = z + mvs


def kernel(zi, y, latent, W_v, b_v, W_p, b_p, E_m):
    zi2 = zi.astype(jnp.int32).reshape(_B, 1)
    rep = (jnp.arange(_R)[:, None] // _S == jnp.arange(_B)[None, :]
           ).astype(jnp.float32)
    wp = W_p.T
    bp = b_p.reshape(1, _ST)
    rkey = jax.random.key(42)
    rd = jnp.stack([
        jax.random.uniform(jax.random.fold_in(rkey, i), (_B, _S),
                           dtype=jnp.float32).reshape(_R)
        for i in range(_T)
    ]).reshape(_T, _R, 1)
    wv = jnp.zeros((_D, _GP), jnp.float32).at[:, :_G].set(W_v.T)
    bv = jnp.full((1, _GP), -1e30, jnp.float32).at[0, :_G].set(b_v)
    y_rep = jnp.repeat(y.astype(jnp.int32).T, _S, axis=1).reshape(_T, _R, 1)

    out_t = pl.pallas_call(
        _body,
        out_shape=jax.ShapeDtypeStruct((_T, 1, _B), jnp.float32),
    )(zi2, latent, rep, wp, bp, E_m, rd, wv, bv, y_rep, rep)

    return out_t.reshape(_T, _B).T


# speedup vs baseline: 3.2096x; 1.2597x over previous
import jax
import jax.numpy as jnp
from jax.experimental import pallas as pl

_B = 64
_T = 16
_S = 10
_D = 128
_G = 1000
_GP = 1024
_ST = 64
_TOT = 4096
_R = _B * _S

_HI = jax.lax.Precision.HIGHEST


def _dot(a, b):
    return jax.lax.dot_general(a, b, (((a.ndim - 1,), (0,)), ((), ())),
                               precision=_HI, preferred_element_type=jnp.float32)


def _cumsum_lanes(x):
    r, n = x.shape
    d = 1
    while d < n:
        x = x + jnp.concatenate(
            [jnp.zeros((r, d), x.dtype), x[:, :-d]], axis=1)
        d *= 2
    return x


def _body(zi_ref, latent_ref, rep_ref, wp_ref, bp_ref, em_ref, rd_ref,
          wv_ref, bv_ref, bvc_ref, emt_ref, wvr_ref, y_ref, out_ref):
    g = _dot(em_ref[...], wv_ref[...]) + bv_ref[...]
    gm = jnp.max(g, axis=-1, keepdims=True)
    lse_state = gm + jnp.log(jnp.sum(jnp.exp(g - gm), axis=-1, keepdims=True))
    gt = _dot(wvr_ref[...], emt_ref[...]) + bvc_ref[...]
    iota_gp = jax.lax.broadcasted_iota(jnp.int32, (_T * _B, _GP), 1)
    onehot_y = (y_ref[...] == iota_gp).astype(jnp.float32)
    h = _dot(onehot_y, gt)
    m_all = h - lse_state.reshape(1, _ST)

    iota_tot = jax.lax.broadcasted_iota(jnp.int32, (_B, _TOT), 1)
    onehot_zi = (zi_ref[...] == iota_tot).astype(jnp.float32)
    z0 = _dot(onehot_zi, latent_ref[...])
    z = _dot(rep_ref[...], z0)
    zold = jnp.zeros_like(z)
    iota_st = jax.lax.broadcasted_iota(jnp.int32, (_R, _ST), 1)
    for i in range(_T):
        mean = jnp.mean(z, axis=-1, keepdims=True)
        c = z - mean
        var = jnp.sum(c * c, axis=-1, keepdims=True) * (1.0 / (_D - 1))
        z = z / (1e-05 + jnp.sqrt(var)) * 0.113
        pref = _dot(jnp.concatenate([zold, z], axis=-1), wp_ref[...]) + bp_ref[...]
        m = jnp.max(pref, axis=-1, keepdims=True)
        e = jnp.exp(pref - m)
        xp = e / jnp.sum(e, axis=-1, keepdims=True)
        xpc = _cumsum_lanes(xp)
        rd = rd_ref[i]
        cnt = jnp.sum((xpc <= rd).astype(jnp.int32), axis=-1, keepdims=True)
        which = jnp.where(cnt >= _ST, 0, cnt)
        onehot = (which == iota_st).astype(jnp.float32)
        mvs = _dot(onehot, em_ref[...])

        mt_exp = _dot(rep_ref[...], m_all[i * _B:(i + 1) * _B, :])
        yp = jnp.sum(onehot * mt_exp, axis=-1, keepdims=True)
        grp = jax.lax.dot_general(yp, rep_ref[...], (((0,), (0,)), ((), ())),
                                  precision=_HI,
                                  preferred_element_type=jnp.float32)
        out_ref[i] = grp * (1.0 / _S)

        zold = z
        z = z + mvs


def kernel(zi, y, latent, W_v, b_v, W_p, b_p, E_m):
    zi2 = zi.astype(jnp.int32).reshape(_B, 1)
    rep = (jnp.arange(_R)[:, None] // _S == jnp.arange(_B)[None, :]
           ).astype(jnp.float32)
    wp = W_p.T
    bp = b_p.reshape(1, _ST)
    rkey = jax.random.key(42)
    rd = jnp.stack([
        jax.random.uniform(jax.random.fold_in(rkey, i), (_B, _S),
                           dtype=jnp.float32).reshape(_R)
        for i in range(_T)
    ]).reshape(_T, _R, 1)
    wv = jnp.zeros((_D, _GP), jnp.float32).at[:, :_G].set(W_v.T)
    bv = jnp.full((1, _GP), -1e30, jnp.float32).at[0, :_G].set(b_v)
    bvc = bv.reshape(_GP, 1)
    emt = E_m.T
    wvr = jnp.zeros((_GP, _D), jnp.float32).at[:_G, :].set(W_v)
    y_t = y.astype(jnp.int32).T.reshape(_T * _B, 1)

    out_t = pl.pallas_call(
        _body,
        out_shape=jax.ShapeDtypeStruct((_T, 1, _B), jnp.float32),
    )(zi2, latent, rep, wp, bp, E_m, rd, wv, bv, bvc, emt, wvr, y_t)

    return out_t.reshape(_T, _B).T


# speedup vs baseline: 3.7020x; 1.1534x over previous
import jax
import jax.numpy as jnp
from jax.experimental import pallas as pl

_B = 64
_T = 16
_S = 10
_D = 128
_G = 1000
_GP = 1024
_ST = 64
_TOT = 4096
_R = _B * _S

_HI = jax.lax.Precision.HIGHEST


def _dot(a, b):
    return jax.lax.dot_general(a, b, (((a.ndim - 1,), (0,)), ((), ())),
                               precision=_HI, preferred_element_type=jnp.float32)


def _cumsum_lanes(x):
    r, n = x.shape
    d = 1
    while d < n:
        x = x + jnp.concatenate(
            [jnp.zeros((r, d), x.dtype), x[:, :-d]], axis=1)
        d *= 2
    return x


def _body(zi_ref, latent_ref, rep_ref, wp_ref, bp_ref, em_ref, rd_ref,
          wv_ref, bv_ref, bvc_ref, emt_ref, wvr_ref, y_ref, tri_ref, out_ref):
    g = _dot(em_ref[...], wv_ref[...]) + bv_ref[...]
    gm = jnp.max(g, axis=-1, keepdims=True)
    lse_state = gm + jnp.log(jnp.sum(jnp.exp(g - gm), axis=-1, keepdims=True))
    gt = _dot(wvr_ref[...], emt_ref[...]) + bvc_ref[...]
    iota_gp = jax.lax.broadcasted_iota(jnp.int32, (_T * _B, _GP), 1)
    onehot_y = (y_ref[...] == iota_gp).astype(jnp.float32)
    h = _dot(onehot_y, gt)
    m_all = h - lse_state.reshape(1, _ST)

    iota_tot = jax.lax.broadcasted_iota(jnp.int32, (_B, _TOT), 1)
    onehot_zi = (zi_ref[...] == iota_tot).astype(jnp.float32)
    z0 = _dot(onehot_zi, latent_ref[...])
    z = _dot(rep_ref[...], z0)
    zold = jnp.zeros_like(z)
    iota_st = jax.lax.broadcasted_iota(jnp.int32, (_R, _ST), 1)
    for i in range(_T):
        mean = jnp.mean(z, axis=-1, keepdims=True)
        c = z - mean
        var = jnp.sum(c * c, axis=-1, keepdims=True) * (1.0 / (_D - 1))
        z = z / (1e-05 + jnp.sqrt(var)) * 0.113
        pref = _dot(jnp.concatenate([zold, z], axis=-1), wp_ref[...]) + bp_ref[...]
        m = jnp.max(pref, axis=-1, keepdims=True)
        e = jnp.exp(pref - m)
        xp = e / jnp.sum(e, axis=-1, keepdims=True)
        xpc = _dot(xp, tri_ref[...])
        rd = rd_ref[i]
        cnt = jnp.sum((xpc <= rd).astype(jnp.int32), axis=-1, keepdims=True)
        which = jnp.where(cnt >= _ST, 0, cnt)
        onehot = (which == iota_st).astype(jnp.float32)
        mvs = _dot(onehot, em_ref[...])

        mt_exp = _dot(rep_ref[...], m_all[i * _B:(i + 1) * _B, :])
        yp = jnp.sum(onehot * mt_exp, axis=-1, keepdims=True)
        grp = jax.lax.dot_general(yp, rep_ref[...], (((0,), (0,)), ((), ())),
                                  precision=_HI,
                                  preferred_element_type=jnp.float32)
        out_ref[i] = grp * (1.0 / _S)

        zold = z
        z = z + mvs


def kernel(zi, y, latent, W_v, b_v, W_p, b_p, E_m):
    zi2 = zi.astype(jnp.int32).reshape(_B, 1)
    rep = (jnp.arange(_R)[:, None] // _S == jnp.arange(_B)[None, :]
           ).astype(jnp.float32)
    wp = W_p.T
    bp = b_p.reshape(1, _ST)
    rkey = jax.random.key(42)
    rd = jnp.stack([
        jax.random.uniform(jax.random.fold_in(rkey, i), (_B, _S),
                           dtype=jnp.float32).reshape(_R)
        for i in range(_T)
    ]).reshape(_T, _R, 1)
    wv = jnp.zeros((_D, _GP), jnp.float32).at[:, :_G].set(W_v.T)
    bv = jnp.full((1, _GP), -1e30, jnp.float32).at[0, :_G].set(b_v)
    bvc = bv.reshape(_GP, 1)
    emt = E_m.T
    wvr = jnp.zeros((_GP, _D), jnp.float32).at[:_G, :].set(W_v)
    y_t = y.astype(jnp.int32).T.reshape(_T * _B, 1)
    tri = (jnp.arange(_ST)[:, None] <= jnp.arange(_ST)[None, :]
           ).astype(jnp.float32)

    out_t = pl.pallas_call(
        _body,
        out_shape=jax.ShapeDtypeStruct((_T, 1, _B), jnp.float32),
    )(zi2, latent, rep, wp, bp, E_m, rd, wv, bv, bvc, emt, wvr, y_t, tri)

    return out_t.reshape(_T, _B).T
